# baseline (speedup 1.0000x reference)
import functools

import jax
import jax.numpy as jnp
from jax.experimental import pallas as pl


def _fused_body(idx_ref, x_ref, W_ref, b_ref, Wd_ref, bd_ref, z_ref, y_ref,
                *, E, L):
    x_t = x_ref[...].astype(jnp.bfloat16)
    ids = idx_ref[...]
    big = jnp.dot(x_t, W_ref[...], preferred_element_type=jnp.float32)
    big = big + b_ref[...]
    acc = jnp.zeros(z_ref.shape, dtype=jnp.float32)
    for e in range(E):
        acc = jnp.where(ids == e, big[:, e * L:(e + 1) * L], acc)
    z_ref[...] = acc
    y_ref[...] = jnp.dot(acc, Wd_ref[...], preferred_element_type=jnp.float32) + bd_ref[0]


def kernel(x, individual_idx, W_enc, b_enc, W_dec, b_dec):
    N, D = x.shape
    E, _, L = W_enc.shape
    TN = 2048
    nb = N // TN
    idx2 = individual_idx.astype(jnp.int32).reshape(N, 1)
    W_all = W_enc.transpose(1, 0, 2).reshape(D, E * L).astype(jnp.bfloat16)
    b_all = b_enc.reshape(1, E * L)

    z, y = pl.pallas_call(
        functools.partial(_fused_body, E=E, L=L),
        grid=(nb,),
        in_specs=[
            pl.BlockSpec((TN, 1), lambda i: (i, 0)),
            pl.BlockSpec((TN, D), lambda i: (i, 0)),
            pl.BlockSpec((D, E * L), lambda i: (0, 0)),
            pl.BlockSpec((1, E * L), lambda i: (0, 0)),
            pl.BlockSpec((L, 1), lambda i: (0, 0)),
            pl.BlockSpec((1,), lambda i: (0,)),
        ],
        out_specs=[
            pl.BlockSpec((TN, L), lambda i: (i, 0)),
            pl.BlockSpec((TN, 1), lambda i: (i, 0)),
        ],
        out_shape=[
            jax.ShapeDtypeStruct((N, L), jnp.float32),
            jax.ShapeDtypeStruct((N, 1), jnp.float32),
        ],
    )(idx2, x, W_all, b_all, W_dec, b_dec)
    return (y, z)
